# Initial kernel scaffold; baseline (speedup 1.0000x reference)
import functools

import jax
import jax.numpy as jnp
from jax import lax
from jax.experimental import pallas as pl
from jax.experimental.pallas import tpu as pltpu
from jax.experimental.pallas import tpu_sc as plsc

_L = 16
_NC = 2
_NS = 16
_NW = _NC * _NS

_N = 100000
_NPAD = 100352
_NPS = _NPAD // _NS

_E = 6400000
_C = 4096
_R = _C // 128
_K = -(-_E // (_NW * _C))
_EPW = _C * _K
_EP = _EPW * _NW


def _sc_body(x_hbm, wts_hbm, src_hbm, dst_hbm, psum_hbm, pcnt_hbm,
             h_vm, src_vm, dst_vm, val_vm, one_vm, sl_vm, w_vm,
             h_sh, sum_sh, cnt_sh):
    cid = lax.axis_index("c")
    sid = lax.axis_index("s")
    wid = cid * _NS + sid
    nbase = sid * _NPS

    pltpu.sync_copy(wts_hbm, w_vm)
    pltpu.sync_copy(x_hbm.at[pl.ds(nbase, _NPS)], sl_vm)

    a1 = [w_vm[j] for j in range(4)]
    c1 = [w_vm[4 + j] for j in range(4)]
    w2 = [w_vm[8 + j] for j in range(4)]
    b2v = w_vm[12]
    zero16 = jnp.zeros((_L,), jnp.float32)
    one16 = jnp.ones((_L,), jnp.float32)

    def mlp_body(i, carry):
        xv = sl_vm[pl.ds(i * _L, _L)]
        acc = b2v
        for j in range(4):
            acc = acc + w2[j] * jnp.maximum(a1[j] * xv + c1[j], zero16)
        sl_vm[pl.ds(i * _L, _L)] = acc
        return carry
    lax.fori_loop(0, _NPS // _L, mlp_body, 0)
    pltpu.sync_copy(sl_vm, h_sh.at[pl.ds(nbase, _NPS)])

    def zero_body(i, carry):
        sl_vm[pl.ds(i * _L, _L)] = zero16
        return carry
    lax.fori_loop(0, _NPS // _L, zero_body, 0)
    pltpu.sync_copy(sl_vm, sum_sh.at[pl.ds(nbase, _NPS)])
    pltpu.sync_copy(sl_vm, cnt_sh.at[pl.ds(nbase, _NPS)])

    def ones_body(r, carry):
        for v in range(128 // _L):
            one_vm[r, pl.ds(v * _L, _L)] = one16
        return carry
    lax.fori_loop(0, _R, ones_body, 0)

    plsc.subcore_barrier()
    pltpu.sync_copy(h_sh, h_vm)

    ebase = wid * _EPW
    rbase = wid * (_EPW // 128)

    def chunk_body(k, carry):
        pltpu.sync_copy(src_hbm.at[pl.ds(ebase + k * _C, _C)], src_vm)
        pltpu.sync_copy(dst_hbm.at[pl.ds(rbase + k * _R, _R)], dst_vm)

        def gather_body(r, gcarry):
            for v in range(128 // _L):
                sv = src_vm[pl.ds(r * 128 + v * _L, _L)]
                val_vm[r, pl.ds(v * _L, _L)] = plsc.load_gather(h_vm, [sv])
            return gcarry
        lax.fori_loop(0, _R, gather_body, 0)

        pltpu.sync_copy(val_vm, sum_sh.at[dst_vm], add=True)
        pltpu.sync_copy(one_vm, cnt_sh.at[dst_vm], add=True)
        return carry
    lax.fori_loop(0, _K, chunk_body, 0)

    plsc.subcore_barrier()

    pltpu.sync_copy(sum_sh.at[pl.ds(nbase, _NPS)],
                    psum_hbm.at[cid, pl.ds(nbase, _NPS)])
    pltpu.sync_copy(cnt_sh.at[pl.ds(nbase, _NPS)],
                    pcnt_hbm.at[cid, pl.ds(nbase, _NPS)])


def _make_sc_call():
    mesh = plsc.VectorSubcoreMesh(core_axis_name="c", subcore_axis_name="s",
                                  num_cores=_NC, num_subcores=_NS)
    return pl.kernel(
        _sc_body,
        out_type=(
            jax.ShapeDtypeStruct((_NC, _NPAD), jnp.float32),
            jax.ShapeDtypeStruct((_NC, _NPAD), jnp.float32),
        ),
        mesh=mesh,
        scratch_types=[
            pltpu.VMEM((_NPAD,), jnp.float32),
            pltpu.VMEM((_C,), jnp.int32),
            pltpu.VMEM((_R, 128), jnp.int32),
            pltpu.VMEM((_R, 128), jnp.float32),
            pltpu.VMEM((_R, 128), jnp.float32),
            pltpu.VMEM((_NPS,), jnp.float32),
            pltpu.VMEM((16, 16), jnp.float32),
            pltpu.VMEM_SHARED((_NPAD,), jnp.float32),
            pltpu.VMEM_SHARED((_NPAD,), jnp.float32),
            pltpu.VMEM_SHARED((_NPAD,), jnp.float32),
        ],
    )


def _combine_body(ps_ref, pc_ref, w_ref, o_ref):
    s = ps_ref[0] + ps_ref[1]
    c = pc_ref[0] + pc_ref[1]
    o_ref[...] = (s / jnp.maximum(c, 1.0)) * w_ref[0, 0]


def kernel(x, edge_index, W1, b1, W2, b2, Wsage):
    xpad = jnp.zeros((_NPAD,), jnp.float32).at[:_N].set(x[:, 0])

    fill = jnp.full((2, _EP - _E), _N, dtype=jnp.int32)
    ei = jnp.concatenate([edge_index, fill], axis=1)
    srcp = ei[0]
    dst2d = ei[1].reshape(_EP // 128, 128)

    wts = jnp.zeros((16, 16), jnp.float32)
    wts = wts.at[0:4].set(jnp.broadcast_to(W1[:, 0:1], (4, 16)))
    wts = wts.at[4:8].set(jnp.broadcast_to(b1[:, None], (4, 16)))
    wts = wts.at[8:12].set(jnp.broadcast_to(W2[0, :, None], (4, 16)))
    wts = wts.at[12].set(jnp.broadcast_to(b2, (16,)))

    psum, pcnt = _make_sc_call()(xpad, wts, srcp, dst2d)

    comb = pl.pallas_call(
        _combine_body,
        out_shape=jax.ShapeDtypeStruct((_NPAD // 128, 128), jnp.float32),
        in_specs=[
            pl.BlockSpec(memory_space=pltpu.VMEM),
            pl.BlockSpec(memory_space=pltpu.VMEM),
            pl.BlockSpec(memory_space=pltpu.SMEM),
        ],
        out_specs=pl.BlockSpec(memory_space=pltpu.VMEM),
    )(psum.reshape(_NC, _NPAD // 128, 128),
      pcnt.reshape(_NC, _NPAD // 128, 128),
      Wsage)

    return comb.reshape(_NPAD)[:_N].reshape(_N, 1)


# speedup vs baseline: 135.5411x; 135.5411x over previous
import jax
import jax.numpy as jnp
from jax import lax
from jax.experimental import pallas as pl
from jax.experimental.pallas import tpu as pltpu
from jax.experimental.pallas import tpu_sc as plsc

_L = 16
_NC = 2
_NS = 16
_NW = _NC * _NS

_N = 100000
_NPAD = 100352
_NPS = _NPAD // _NS

_E = 6400000
_C = 4096
_R = _C // 128
_K = -(-_E // (_NW * _C))
_EPW = _C * _K
_EP = _EPW * _NW


def _mlp_body(x_ref, w1_ref, b1_ref, w2_ref, b2_ref, h_ref):
    xv = x_ref[...]
    acc = jnp.zeros_like(xv) + b2_ref[0]
    for j in range(4):
        acc = acc + w2_ref[j] * jnp.maximum(w1_ref[j] * xv + b1_ref[j], 0.0)
    h_ref[...] = acc


def _sc_body(h_hbm, z_hbm, src_hbm, dst_hbm, psum_hbm, pcnt_hbm,
             h_vm, src_vm, dst_vm, val_vm, one_vm,
             sum_sh, cnt_sh):
    cid = lax.axis_index("c")
    sid = lax.axis_index("s")
    wid = cid * _NS + sid
    nbase = sid * _NPS

    pltpu.sync_copy(h_hbm, h_vm)
    pltpu.sync_copy(z_hbm.at[pl.ds(nbase, _NPS)], sum_sh.at[pl.ds(nbase, _NPS)])
    pltpu.sync_copy(z_hbm.at[pl.ds(nbase, _NPS)], cnt_sh.at[pl.ds(nbase, _NPS)])

    one16 = jnp.ones((_L,), jnp.float32)
    for v in range(128 // _L):
        one_vm[pl.ds(v * _L, _L)] = one16

    plsc.subcore_barrier()

    ebase = wid * _EPW
    rbase = wid * (_EPW // 128)

    def chunk_body(k, carry):
        pltpu.sync_copy(src_hbm.at[pl.ds(ebase + k * _C, _C)], src_vm)
        pltpu.sync_copy(dst_hbm.at[pl.ds(rbase + k * _R, _R)], dst_vm)

        def gather_body(r, gcarry):
            for v in range(128 // _L):
                sv = src_vm[pl.ds(r * 128 + v * _L, _L)]
                val_vm[r, pl.ds(v * _L, _L)] = plsc.load_gather(h_vm, [sv])
            return gcarry
        lax.fori_loop(0, _R, gather_body, 0)

        def scat_body(r, scarry):
            pltpu.sync_copy(val_vm.at[r], sum_sh.at[dst_vm.at[r]], add=True)
            pltpu.sync_copy(one_vm, cnt_sh.at[dst_vm.at[r]], add=True)
            return scarry
        lax.fori_loop(0, _R, scat_body, 0)
        return carry
    lax.fori_loop(0, _K, chunk_body, 0)

    plsc.subcore_barrier()

    pltpu.sync_copy(sum_sh.at[pl.ds(nbase, _NPS)],
                    psum_hbm.at[cid, pl.ds(nbase, _NPS)])
    pltpu.sync_copy(cnt_sh.at[pl.ds(nbase, _NPS)],
                    pcnt_hbm.at[cid, pl.ds(nbase, _NPS)])


def _make_sc_call():
    mesh = plsc.VectorSubcoreMesh(core_axis_name="c", subcore_axis_name="s",
                                  num_cores=_NC, num_subcores=_NS)
    return pl.kernel(
        _sc_body,
        out_type=(
            jax.ShapeDtypeStruct((_NC, _NPAD), jnp.float32),
            jax.ShapeDtypeStruct((_NC, _NPAD), jnp.float32),
        ),
        mesh=mesh,
        compiler_params=pltpu.CompilerParams(needs_layout_passes=False),
        scratch_types=[
            pltpu.VMEM((_NPAD,), jnp.float32),
            pltpu.VMEM((_C,), jnp.int32),
            pltpu.VMEM((_R, 128), jnp.int32),
            pltpu.VMEM((_R, 128), jnp.float32),
            pltpu.VMEM((128,), jnp.float32),
            pltpu.VMEM_SHARED((_NPAD,), jnp.float32),
            pltpu.VMEM_SHARED((_NPAD,), jnp.float32),
        ],
    )


def _combine_body(ps_ref, pc_ref, w_ref, o_ref):
    s = ps_ref[0] + ps_ref[1]
    c = pc_ref[0] + pc_ref[1]
    o_ref[...] = (s / jnp.maximum(c, 1.0)) * w_ref[0]


def kernel(x, edge_index, W1, b1, W2, b2, Wsage):
    xpad = jnp.zeros((_NPAD,), jnp.float32).at[:_N].set(x[:, 0])

    fill = jnp.full((2, _EP - _E), _N, dtype=jnp.int32)
    ei = jnp.concatenate([edge_index, fill], axis=1)
    srcp = ei[0]
    dst2d = ei[1].reshape(_EP // 128, 128)

    h = pl.pallas_call(
        _mlp_body,
        out_shape=jax.ShapeDtypeStruct((_NPAD // 128, 128), jnp.float32),
        in_specs=[
            pl.BlockSpec(memory_space=pltpu.VMEM),
            pl.BlockSpec(memory_space=pltpu.SMEM),
            pl.BlockSpec(memory_space=pltpu.SMEM),
            pl.BlockSpec(memory_space=pltpu.SMEM),
            pl.BlockSpec(memory_space=pltpu.SMEM),
        ],
        out_specs=pl.BlockSpec(memory_space=pltpu.VMEM),
    )(xpad.reshape(_NPAD // 128, 128), W1[:, 0], b1, W2[0], b2)

    zeros = jnp.zeros((_NPAD,), jnp.float32)
    psum, pcnt = _make_sc_call()(h.reshape(_NPAD), zeros, srcp, dst2d)

    comb = pl.pallas_call(
        _combine_body,
        out_shape=jax.ShapeDtypeStruct((_NPAD // 128, 128), jnp.float32),
        in_specs=[
            pl.BlockSpec(memory_space=pltpu.VMEM),
            pl.BlockSpec(memory_space=pltpu.VMEM),
            pl.BlockSpec(memory_space=pltpu.SMEM),
        ],
        out_specs=pl.BlockSpec(memory_space=pltpu.VMEM),
    )(psum.reshape(_NC, _NPAD // 128, 128),
      pcnt.reshape(_NC, _NPAD // 128, 128),
      Wsage[0])

    return comb.reshape(_NPAD)[:_N].reshape(_N, 1)


# speedup vs baseline: 194.1136x; 1.4321x over previous
import jax
import jax.numpy as jnp
from jax import lax
from jax.experimental import pallas as pl
from jax.experimental.pallas import tpu as pltpu
from jax.experimental.pallas import tpu_sc as plsc

_L = 16
_NC = 2
_NS = 16
_NW = _NC * _NS

_N = 100000
_NPAD = 100352
_NPS = _NPAD // _NS

_E = 6400000
_C = 4096
_R = _C // 128
_G = 8
_K = -(-_E // (_NW * _C))
_EPW = _C * _K
_EP = _EPW * _NW


def _mlp_body(x_ref, w1_ref, b1_ref, w2_ref, b2_ref, h_ref):
    xv = x_ref[...]
    acc = jnp.zeros_like(xv) + b2_ref[0]
    for j in range(4):
        acc = acc + w2_ref[j] * jnp.maximum(w1_ref[j] * xv + b1_ref[j], 0.0)
    h_ref[...] = acc


def _sc_body(h_hbm, z_hbm, src_hbm, dst_hbm, psum_hbm, pcnt_hbm,
             h_vm, src_vm, dst_vm, val_vm, one_vm,
             sum_sh, cnt_sh, sem):
    cid = lax.axis_index("c")
    sid = lax.axis_index("s")
    wid = cid * _NS + sid
    nbase = sid * _NPS

    pltpu.sync_copy(h_hbm, h_vm)
    pltpu.sync_copy(z_hbm.at[pl.ds(nbase, _NPS)], sum_sh.at[pl.ds(nbase, _NPS)])
    pltpu.sync_copy(z_hbm.at[pl.ds(nbase, _NPS)], cnt_sh.at[pl.ds(nbase, _NPS)])

    one16 = jnp.ones((_L,), jnp.float32)
    for v in range(128 // _L):
        one_vm[pl.ds(v * _L, _L)] = one16

    plsc.subcore_barrier()

    ebase = wid * _EPW
    rbase = wid * (_EPW // 128)

    def chunk_body(k, carry):
        pltpu.sync_copy(src_hbm.at[pl.ds(ebase + k * _C, _C)], src_vm)
        pltpu.sync_copy(dst_hbm.at[pl.ds(rbase + k * _R, _R)], dst_vm)

        def gather_body(r, gcarry):
            for v in range(128 // _L):
                sv = src_vm[pl.ds(r * 128 + v * _L, _L)]
                val_vm[r, pl.ds(v * _L, _L)] = plsc.load_gather(h_vm, [sv])
            return gcarry
        lax.fori_loop(0, _R, gather_body, 0)

        def scat_group(g, scarry):
            descs = []
            for j in range(_G):
                r = g * _G + j
                descs.append(pltpu.async_copy(
                    val_vm.at[r], sum_sh.at[dst_vm.at[r]], sem, add=True))
                descs.append(pltpu.async_copy(
                    one_vm, cnt_sh.at[dst_vm.at[r]], sem, add=True))
            for d in descs:
                d.wait()
            return scarry
        lax.fori_loop(0, _R // _G, scat_group, 0)
        return carry
    lax.fori_loop(0, _K, chunk_body, 0)

    plsc.subcore_barrier()

    pltpu.sync_copy(sum_sh.at[pl.ds(nbase, _NPS)],
                    psum_hbm.at[cid, pl.ds(nbase, _NPS)])
    pltpu.sync_copy(cnt_sh.at[pl.ds(nbase, _NPS)],
                    pcnt_hbm.at[cid, pl.ds(nbase, _NPS)])


def _make_sc_call():
    mesh = plsc.VectorSubcoreMesh(core_axis_name="c", subcore_axis_name="s",
                                  num_cores=_NC, num_subcores=_NS)
    return pl.kernel(
        _sc_body,
        out_type=(
            jax.ShapeDtypeStruct((_NC, _NPAD), jnp.float32),
            jax.ShapeDtypeStruct((_NC, _NPAD), jnp.float32),
        ),
        mesh=mesh,
        compiler_params=pltpu.CompilerParams(needs_layout_passes=False),
        scratch_types=[
            pltpu.VMEM((_NPAD,), jnp.float32),
            pltpu.VMEM((_C,), jnp.int32),
            pltpu.VMEM((_R, 128), jnp.int32),
            pltpu.VMEM((_R, 128), jnp.float32),
            pltpu.VMEM((128,), jnp.float32),
            pltpu.VMEM_SHARED((_NPAD,), jnp.float32),
            pltpu.VMEM_SHARED((_NPAD,), jnp.float32),
            pltpu.SemaphoreType.DMA,
        ],
    )


def _combine_body(ps_ref, pc_ref, w_ref, o_ref):
    s = ps_ref[0] + ps_ref[1]
    c = pc_ref[0] + pc_ref[1]
    o_ref[...] = (s / jnp.maximum(c, 1.0)) * w_ref[0]


def kernel(x, edge_index, W1, b1, W2, b2, Wsage):
    xpad = jnp.zeros((_NPAD,), jnp.float32).at[:_N].set(x[:, 0])

    fill = jnp.full((2, _EP - _E), _N, dtype=jnp.int32)
    ei = jnp.concatenate([edge_index, fill], axis=1)
    srcp = ei[0]
    dst2d = ei[1].reshape(_EP // 128, 128)

    h = pl.pallas_call(
        _mlp_body,
        out_shape=jax.ShapeDtypeStruct((_NPAD // 128, 128), jnp.float32),
        in_specs=[
            pl.BlockSpec(memory_space=pltpu.VMEM),
            pl.BlockSpec(memory_space=pltpu.SMEM),
            pl.BlockSpec(memory_space=pltpu.SMEM),
            pl.BlockSpec(memory_space=pltpu.SMEM),
            pl.BlockSpec(memory_space=pltpu.SMEM),
        ],
        out_specs=pl.BlockSpec(memory_space=pltpu.VMEM),
    )(xpad.reshape(_NPAD // 128, 128), W1[:, 0], b1, W2[0], b2)

    zeros = jnp.zeros((_NPAD,), jnp.float32)
    psum, pcnt = _make_sc_call()(h.reshape(_NPAD), zeros, srcp, dst2d)

    comb = pl.pallas_call(
        _combine_body,
        out_shape=jax.ShapeDtypeStruct((_NPAD // 128, 128), jnp.float32),
        in_specs=[
            pl.BlockSpec(memory_space=pltpu.VMEM),
            pl.BlockSpec(memory_space=pltpu.VMEM),
            pl.BlockSpec(memory_space=pltpu.SMEM),
        ],
        out_specs=pl.BlockSpec(memory_space=pltpu.VMEM),
    )(psum.reshape(_NC, _NPAD // 128, 128),
      pcnt.reshape(_NC, _NPAD // 128, 128),
      Wsage[0])

    return comb.reshape(_NPAD)[:_N].reshape(_N, 1)


# speedup vs baseline: 274.1118x; 1.4121x over previous
import jax
import jax.numpy as jnp
from jax import lax
from jax.experimental import pallas as pl
from jax.experimental.pallas import tpu as pltpu
from jax.experimental.pallas import tpu_sc as plsc

_L = 16
_NC = 2
_NS = 16
_NW = _NC * _NS

_N = 100000
_NPAD = 100352
_NPS = _NPAD // _NS

_E = 6400000
_C = 2048
_R = _C // 128
_G = 8
_K = 98
_EPW = _C * _K
_EP = _EPW * _NW


def _mlp_body(x_ref, w1_ref, b1_ref, w2_ref, b2_ref, h_ref):
    xv = x_ref[...]
    acc = jnp.zeros_like(xv) + b2_ref[0]
    for j in range(4):
        acc = acc + w2_ref[j] * jnp.maximum(w1_ref[j] * xv + b1_ref[j], 0.0)
    h_ref[...] = acc


def _sc_body(h_hbm, z_hbm, src_hbm, dst_hbm, psum_hbm, pcnt_hbm,
             h_vm, src_vm, dst_vm, val_vm, one_vm,
             sum_sh, cnt_sh, sem_in, sem_sc):
    cid = lax.axis_index("c")
    sid = lax.axis_index("s")
    wid = cid * _NS + sid
    nbase = sid * _NPS

    pltpu.sync_copy(h_hbm, h_vm)
    pltpu.sync_copy(z_hbm.at[pl.ds(nbase, _NPS)], sum_sh.at[pl.ds(nbase, _NPS)])
    pltpu.sync_copy(z_hbm.at[pl.ds(nbase, _NPS)], cnt_sh.at[pl.ds(nbase, _NPS)])

    one16 = jnp.ones((_L,), jnp.float32)
    for v in range(128 // _L):
        one_vm[pl.ds(v * _L, _L)] = one16

    plsc.subcore_barrier()

    ebase = wid * _EPW
    rbase = wid * (_EPW // 128)

    def fire_input(k):
        b2 = lax.rem(k, 2)
        b3 = lax.rem(k, 3)
        pltpu.async_copy(src_hbm.at[pl.ds(ebase + k * _C, _C)],
                         src_vm.at[b2], sem_in)
        pltpu.async_copy(dst_hbm.at[pl.ds(rbase + k * _R, _R)],
                         dst_vm.at[b3], sem_in)

    def wait_input(k):
        b2 = lax.rem(k, 2)
        b3 = lax.rem(k, 3)
        pltpu.make_async_copy(src_hbm.at[pl.ds(ebase + k * _C, _C)],
                              src_vm.at[b2], sem_in).wait()
        pltpu.make_async_copy(dst_hbm.at[pl.ds(rbase + k * _R, _R)],
                              dst_vm.at[b3], sem_in).wait()

    def drain_scatters(k):
        b3 = lax.rem(k, 3)

        def drain_group(g, dcarry):
            for j in range(_G):
                r = g * _G + j
                pltpu.make_async_copy(val_vm.at[b3, r],
                                      sum_sh.at[dst_vm.at[b3, r]],
                                      sem_sc).wait()
                pltpu.make_async_copy(one_vm,
                                      cnt_sh.at[dst_vm.at[b3, r]],
                                      sem_sc).wait()
            return dcarry
        lax.fori_loop(0, _R // _G, drain_group, 0)

    fire_input(0)

    def chunk_body(k, carry):
        b2 = lax.rem(k, 2)
        b3 = lax.rem(k, 3)
        wait_input(k)

        @pl.when(k >= 2)
        def _():
            drain_scatters(k - 2)

        @pl.when(k + 1 < _K)
        def _():
            fire_input(k + 1)

        def work_group(g, gcarry):
            for j in range(_G):
                r = g * _G + j
                for v in range(128 // _L):
                    sv = src_vm[b2, pl.ds(r * 128 + v * _L, _L)]
                    val_vm[b3, r, pl.ds(v * _L, _L)] = \
                        plsc.load_gather(h_vm, [sv])
                pltpu.async_copy(val_vm.at[b3, r],
                                 sum_sh.at[dst_vm.at[b3, r]],
                                 sem_sc, add=True)
                pltpu.async_copy(one_vm,
                                 cnt_sh.at[dst_vm.at[b3, r]],
                                 sem_sc, add=True)
            return gcarry
        lax.fori_loop(0, _R // _G, work_group, 0)
        return carry
    lax.fori_loop(0, _K, chunk_body, 0)

    drain_scatters(_K - 2)
    drain_scatters(_K - 1)

    plsc.subcore_barrier()

    pltpu.sync_copy(sum_sh.at[pl.ds(nbase, _NPS)],
                    psum_hbm.at[cid, pl.ds(nbase, _NPS)])
    pltpu.sync_copy(cnt_sh.at[pl.ds(nbase, _NPS)],
                    pcnt_hbm.at[cid, pl.ds(nbase, _NPS)])


def _make_sc_call():
    mesh = plsc.VectorSubcoreMesh(core_axis_name="c", subcore_axis_name="s",
                                  num_cores=_NC, num_subcores=_NS)
    return pl.kernel(
        _sc_body,
        out_type=(
            jax.ShapeDtypeStruct((_NC, _NPAD), jnp.float32),
            jax.ShapeDtypeStruct((_NC, _NPAD), jnp.float32),
        ),
        mesh=mesh,
        compiler_params=pltpu.CompilerParams(needs_layout_passes=False),
        scratch_types=[
            pltpu.VMEM((_NPAD,), jnp.float32),
            pltpu.VMEM((2, _C), jnp.int32),
            pltpu.VMEM((3, _R, 128), jnp.int32),
            pltpu.VMEM((3, _R, 128), jnp.float32),
            pltpu.VMEM((128,), jnp.float32),
            pltpu.VMEM_SHARED((_NPAD,), jnp.float32),
            pltpu.VMEM_SHARED((_NPAD,), jnp.float32),
            pltpu.SemaphoreType.DMA,
            pltpu.SemaphoreType.DMA,
        ],
    )


def _combine_body(ps_ref, pc_ref, w_ref, o_ref):
    s = ps_ref[0] + ps_ref[1]
    c = pc_ref[0] + pc_ref[1]
    o_ref[...] = (s / jnp.maximum(c, 1.0)) * w_ref[0]


def kernel(x, edge_index, W1, b1, W2, b2, Wsage):
    xpad = jnp.zeros((_NPAD,), jnp.float32).at[:_N].set(x[:, 0])

    fill = jnp.full((2, _EP - _E), _N, dtype=jnp.int32)
    ei = jnp.concatenate([edge_index, fill], axis=1)
    srcp = ei[0]
    dst2d = ei[1].reshape(_EP // 128, 128)

    h = pl.pallas_call(
        _mlp_body,
        out_shape=jax.ShapeDtypeStruct((_NPAD // 128, 128), jnp.float32),
        in_specs=[
            pl.BlockSpec(memory_space=pltpu.VMEM),
            pl.BlockSpec(memory_space=pltpu.SMEM),
            pl.BlockSpec(memory_space=pltpu.SMEM),
            pl.BlockSpec(memory_space=pltpu.SMEM),
            pl.BlockSpec(memory_space=pltpu.SMEM),
        ],
        out_specs=pl.BlockSpec(memory_space=pltpu.VMEM),
    )(xpad.reshape(_NPAD // 128, 128), W1[:, 0], b1, W2[0], b2)

    zeros = jnp.zeros((_NPAD,), jnp.float32)
    psum, pcnt = _make_sc_call()(h.reshape(_NPAD), zeros, srcp, dst2d)

    comb = pl.pallas_call(
        _combine_body,
        out_shape=jax.ShapeDtypeStruct((_NPAD // 128, 128), jnp.float32),
        in_specs=[
            pl.BlockSpec(memory_space=pltpu.VMEM),
            pl.BlockSpec(memory_space=pltpu.VMEM),
            pl.BlockSpec(memory_space=pltpu.SMEM),
        ],
        out_specs=pl.BlockSpec(memory_space=pltpu.VMEM),
    )(psum.reshape(_NC, _NPAD // 128, 128),
      pcnt.reshape(_NC, _NPAD // 128, 128),
      Wsage[0])

    return comb.reshape(_NPAD)[:_N].reshape(_N, 1)
